# baseline (speedup 1.0000x reference)
import jax
import jax.numpy as jnp
from jax import lax
from jax.experimental import pallas as pl
from jax.experimental.pallas import tpu as pltpu
from jax.experimental.pallas import tpu_sc as plsc

_B = 4096
_D = 128
_NNEG = 64
_REG = 0.01

_info = plsc.get_sparse_core_info()
_NC = _info.num_cores
_NS = _info.num_subcores
_L = _info.num_lanes
_NW = _NC * _NS
_BPW = _B // _NW
_NV = _D // _L
_CH = 32
_NCH = _BPW // _CH


def _sc_body(flat_hbm, rels_hbm, nidx_hbm, relw_hbm,
             neg_out, relrow_out,
             flat_sh, rels_v, nidx_c, relrow_c,
             buf0, buf1, scores_c,
             sem0, sem1, sem_rel):
    sid = lax.axis_index("s")
    wid = sid * _NC + lax.axis_index("c")
    base = wid * _BPW

    pltpu.sync_copy(rels_hbm.at[pl.ds(base, _BPW)], rels_v)

    rows_per_sub = 2 * _B // _NS
    pltpu.sync_copy(flat_hbm.at[pl.ds(sid * rows_per_sub, rows_per_sub)],
                    flat_sh.at[pl.ds(sid * rows_per_sub, rows_per_sub)])
    plsc.subcore_barrier()

    bufs = (buf0, buf1)
    sems = (sem0, sem1)

    zero = jnp.zeros((_L,), jnp.float32)
    lane = lax.iota(jnp.int32, _L)
    rots = [((lane + k) & (_L - 1)).reshape(_L, 1) for k in (8, 4, 2, 1)]
    _dnums = lax.GatherDimensionNumbers(
        offset_dims=(), collapsed_slice_dims=(0,), start_index_map=(0,))

    def lane_sum(x):
        for perm in rots:
            x = x + lax.gather(x, perm, _dnums, (1,),
                               mode=lax.GatherScatterMode.PROMISE_IN_BOUNDS)
        return x

    def cp(ii, par):
        src = flat_sh if par == 0 else flat_hbm
        return pltpu.make_async_copy(src.at[nidx_c.at[ii]],
                                     bufs[par], sems[par])

    for ch in range(_NCH):
        off = base + ch * _CH
        pltpu.sync_copy(nidx_hbm.at[pl.ds(off, _CH)], nidx_c)
        rel_cp = pltpu.async_copy(
            relw_hbm.at[rels_v.at[pl.ds(ch * _CH, _CH)]], relrow_c, sem_rel)
        cp(0, 0).start()
        cp(1, 1).start()
        rel_cp.wait()

        @pl.loop(0, _CH, step=2)
        def _i_loop(i):
            for par in range(2):
                ii = i + par
                cp(ii, par).wait()
                buf = bufs[par]

                relv = [relrow_c[ii, pl.ds(v * _L, _L)] for v in range(_NV)]

                for c in range(_NNEG // _L):
                    @plsc.parallel_loop(0, _L, carry=zero, unroll=4)
                    def pending(n, pending):
                        nn = 2 * (c * _L + n)
                        q = [buf[nn, pl.ds(v * _L, _L)]
                             * buf[nn + 1, pl.ds(v * _L, _L)]
                             * relv[v] for v in range(_NV)]
                        s = (((q[0] + q[1]) + (q[2] + q[3]))
                             + ((q[4] + q[5]) + (q[6] + q[7])))
                        return jnp.where(lane == n, lane_sum(s), pending)

                    scores_c[ii, pl.ds(c * _L, _L)] = pending

                @pl.when(ii + 2 < _CH)
                def _prefetch():
                    cp(ii + 2, par).start()

        pltpu.sync_copy(scores_c, neg_out.at[pl.ds(off, _CH)])
        pltpu.sync_copy(relrow_c, relrow_out.at[pl.ds(off, _CH)])


def _sc_scores(flat, rels, nidx, relw):
    mesh = plsc.VectorSubcoreMesh(core_axis_name="c", subcore_axis_name="s")
    return pl.kernel(
        _sc_body,
        out_type=(
            jax.ShapeDtypeStruct((_B, _NNEG), jnp.float32),
            jax.ShapeDtypeStruct((_B, _D), jnp.float32),
        ),
        mesh=mesh,
        scratch_types=[
            pltpu.VMEM_SHARED((2 * _B, _D), jnp.float32),
            pltpu.VMEM((_BPW,), jnp.int32),
            pltpu.VMEM((_CH, 2 * _NNEG), jnp.int32),
            pltpu.VMEM((_CH, _D), jnp.float32),
            pltpu.VMEM((2 * _NNEG, _D), jnp.float32),
            pltpu.VMEM((2 * _NNEG, _D), jnp.float32),
            pltpu.VMEM((_CH, _NNEG), jnp.float32),
            pltpu.SemaphoreType.DMA,
            pltpu.SemaphoreType.DMA,
            pltpu.SemaphoreType.DMA,
        ],
    )(flat, rels, nidx, relw)


def _log_sigmoid(x):
    return jnp.minimum(x, 0.0) - jnp.log1p(jnp.exp(-jnp.abs(x)))


def _tc_body(neg_ref, relrow_ref, ent_ref, out_ref):
    neg = neg_ref[...]
    rel = relrow_ref[...]
    ent = ent_ref[...]
    heads = ent[:, 0, :]
    tails = ent[:, 1, :]
    pos = jnp.sum(heads * rel * tails, axis=-1)
    neg_loss = -jnp.sum(_log_sigmoid(-neg)) / (_B * _NNEG)
    pos_loss = -jnp.sum(_log_sigmoid(pos)) / _B
    model_loss = (pos_loss + neg_loss) * 0.5
    ent_sq = jnp.sum(ent * ent) / (_B * _D)
    rel_sq = jnp.sum(rel * rel) / (_B * _D)
    reg = _REG * ((ent_sq + rel_sq) / 3.0)
    out_ref[...] = jnp.full((1, 1), 0.0, jnp.float32) + model_loss + reg


def _tc_finish(neg_scores, relrows, ent_embs):
    out = pl.pallas_call(
        _tc_body,
        out_shape=jax.ShapeDtypeStruct((1, 1), jnp.float32),
    )(neg_scores, relrows, ent_embs)
    return out[0, 0]


def kernel(ent_embs, rels, neg_idx, rel_emb_weight):
    ent = ent_embs.astype(jnp.float32)
    flat = ent.reshape(2 * _B, _D)
    rels1 = rels.reshape(_B).astype(jnp.int32)
    nidx = neg_idx.astype(jnp.int32).reshape(_B, 2 * _NNEG)
    relw = rel_emb_weight.astype(jnp.float32)
    neg_scores, relrows = _sc_scores(flat, rels1, nidx, relw)
    return _tc_finish(neg_scores, relrows, ent)


# speedup vs baseline: 1.2659x; 1.2659x over previous
import jax
import jax.numpy as jnp
from jax import lax
from jax.experimental import pallas as pl
from jax.experimental.pallas import tpu as pltpu
from jax.experimental.pallas import tpu_sc as plsc

_B = 4096
_D = 128
_NNEG = 64
_REG = 0.01

_info = plsc.get_sparse_core_info()
_NC = _info.num_cores
_NS = _info.num_subcores
_L = _info.num_lanes
_NW = _NC * _NS
_BPW = _B // _NW
_NV = _D // _L
_CH = 32
_NCH = _BPW // _CH


def _sc_body(flat_hbm, rels_hbm, nidx_hbm, relw_hbm,
             neg_out, relrow_out,
             flat_sh, rels_v, nidx_c, relrow_c,
             buf0, buf1, buf2, scores_c,
             sem0, sem1, sem2, sem_rel):
    sid = lax.axis_index("s")
    wid = sid * _NC + lax.axis_index("c")
    base = wid * _BPW

    pltpu.sync_copy(rels_hbm.at[pl.ds(base, _BPW)], rels_v)

    rows_per_sub = 2 * _B // _NS
    pltpu.sync_copy(flat_hbm.at[pl.ds(sid * rows_per_sub, rows_per_sub)],
                    flat_sh.at[pl.ds(sid * rows_per_sub, rows_per_sub)])
    plsc.subcore_barrier()

    bufs = (buf0, buf1, buf2)
    sems = (sem0, sem1, sem2)

    zero = jnp.zeros((_L,), jnp.float32)
    lane = lax.iota(jnp.int32, _L)
    rots = [((lane + k) & (_L - 1)).reshape(_L, 1) for k in (8, 4, 2, 1)]
    _dnums = lax.GatherDimensionNumbers(
        offset_dims=(), collapsed_slice_dims=(0,), start_index_map=(0,))

    def lane_sum(x):
        for perm in rots:
            x = x + lax.gather(x, perm, _dnums, (1,),
                               mode=lax.GatherScatterMode.PROMISE_IN_BOUNDS)
        return x

    def cp(ii, par):
        return pltpu.make_async_copy(flat_sh.at[nidx_c.at[ii]],
                                     bufs[par], sems[par])

    def compute_one(ii, par):
        cp(ii, par).wait()
        buf = bufs[par]

        relv = [relrow_c[ii, pl.ds(v * _L, _L)] for v in range(_NV)]

        for c in range(_NNEG // _L):
            @plsc.parallel_loop(0, _L, carry=zero, unroll=4)
            def pending(n, pending):
                nn = 2 * (c * _L + n)
                q = [buf[nn, pl.ds(v * _L, _L)]
                     * buf[nn + 1, pl.ds(v * _L, _L)]
                     * relv[v] for v in range(_NV)]
                s = (((q[0] + q[1]) + (q[2] + q[3]))
                     + ((q[4] + q[5]) + (q[6] + q[7])))
                return jnp.where(lane == n, lane_sum(s), pending)

            scores_c[ii, pl.ds(c * _L, _L)] = pending

        @pl.when(ii + 3 < _CH)
        def _prefetch():
            cp(ii + 3, par).start()

    for ch in range(_NCH):
        off = base + ch * _CH
        pltpu.sync_copy(nidx_hbm.at[pl.ds(off, _CH)], nidx_c)
        rel_cp = pltpu.async_copy(
            relw_hbm.at[rels_v.at[pl.ds(ch * _CH, _CH)]], relrow_c, sem_rel)
        cp(0, 0).start()
        cp(1, 1).start()
        cp(2, 2).start()
        rel_cp.wait()

        @pl.loop(0, _CH + 1, step=3)
        def _i_loop(i):
            for par in range(3):
                ii = i + par

                @pl.when(ii < _CH)
                def _do_one():
                    compute_one(ii, par)

        pltpu.sync_copy(scores_c, neg_out.at[pl.ds(off, _CH)])
        pltpu.sync_copy(relrow_c, relrow_out.at[pl.ds(off, _CH)])


def _sc_scores(flat, rels, nidx, relw):
    mesh = plsc.VectorSubcoreMesh(core_axis_name="c", subcore_axis_name="s")
    return pl.kernel(
        _sc_body,
        out_type=(
            jax.ShapeDtypeStruct((_B, _NNEG), jnp.float32),
            jax.ShapeDtypeStruct((_B, _D), jnp.float32),
        ),
        mesh=mesh,
        scratch_types=[
            pltpu.VMEM_SHARED((2 * _B, _D), jnp.float32),
            pltpu.VMEM((_BPW,), jnp.int32),
            pltpu.VMEM((_CH, 2 * _NNEG), jnp.int32),
            pltpu.VMEM((_CH, _D), jnp.float32),
            pltpu.VMEM((2 * _NNEG, _D), jnp.float32),
            pltpu.VMEM((2 * _NNEG, _D), jnp.float32),
            pltpu.VMEM((2 * _NNEG, _D), jnp.float32),
            pltpu.VMEM((_CH, _NNEG), jnp.float32),
            pltpu.SemaphoreType.DMA,
            pltpu.SemaphoreType.DMA,
            pltpu.SemaphoreType.DMA,
            pltpu.SemaphoreType.DMA,
        ],
    )(flat, rels, nidx, relw)


def _log_sigmoid(x):
    return jnp.minimum(x, 0.0) - jnp.log1p(jnp.exp(-jnp.abs(x)))


def _tc_body(neg_ref, relrow_ref, ent_ref, out_ref):
    neg = neg_ref[...]
    rel = relrow_ref[...]
    ent = ent_ref[...]
    heads = ent[:, 0, :]
    tails = ent[:, 1, :]
    pos = jnp.sum(heads * rel * tails, axis=-1)
    neg_loss = -jnp.sum(_log_sigmoid(-neg)) / (_B * _NNEG)
    pos_loss = -jnp.sum(_log_sigmoid(pos)) / _B
    model_loss = (pos_loss + neg_loss) * 0.5
    ent_sq = jnp.sum(ent * ent) / (_B * _D)
    rel_sq = jnp.sum(rel * rel) / (_B * _D)
    reg = _REG * ((ent_sq + rel_sq) / 3.0)
    out_ref[...] = jnp.full((1, 1), 0.0, jnp.float32) + model_loss + reg


def _tc_finish(neg_scores, relrows, ent_embs):
    out = pl.pallas_call(
        _tc_body,
        out_shape=jax.ShapeDtypeStruct((1, 1), jnp.float32),
    )(neg_scores, relrows, ent_embs)
    return out[0, 0]


def kernel(ent_embs, rels, neg_idx, rel_emb_weight):
    ent = ent_embs.astype(jnp.float32)
    flat = ent.reshape(2 * _B, _D)
    rels1 = rels.reshape(_B).astype(jnp.int32)
    nidx = neg_idx.astype(jnp.int32).reshape(_B, 2 * _NNEG)
    relw = rel_emb_weight.astype(jnp.float32)
    neg_scores, relrows = _sc_scores(flat, rels1, nidx, relw)
    return _tc_finish(neg_scores, relrows, ent)
